# Initial kernel scaffold; baseline (speedup 1.0000x reference)
import jax
import jax.numpy as jnp
from jax.experimental import pallas as pl


def kernel(x, edge_index, W1, att_src1, att_dst1, b1, W2, att_src2, att_dst2, b2):
    raise NotImplementedError("write your pallas kernel here")


# speedup vs baseline: 40.1343x; 40.1343x over previous
import functools

import jax
import jax.numpy as jnp
from jax import lax
from jax.experimental import pallas as pl
from jax.experimental.pallas import tpu as pltpu
from jax.experimental.pallas import tpu_sc as plsc

N_NODES = 10000
NPAD = 10240
D_IN = 128
NW = 32
K = 128
ROWS_PER_TILE = NPAD // 16

_f32 = jnp.float32


def _edge_layout(n_edges):
    etot = ((n_edges + N_NODES + NW * K - 1) // (NW * K)) * (NW * K)
    epw = etot // NW
    return etot, epw, epw // K


def _pre_body(x_ref, wall_ref, s_ref, d_ref):
    m = jnp.dot(x_ref[...], wall_ref[...], preferred_element_type=_f32)
    s_ref[...] = m[:, :128]
    d_ref[...] = m[:, 128:]


def _mid_body(a0_ref, a1_ref, b1_ref, w2e_ref, s2_ref, d2_ref):
    num = a0_ref[:, :64] + a1_ref[:, :64]
    den = a0_ref[:, 64:] + a1_ref[:, 64:]
    h = jnp.maximum(num / (den + 1e-16) + b1_ref[...], 0.0)
    m2 = jnp.dot(h, w2e_ref[...], preferred_element_type=_f32)
    col = lax.broadcasted_iota(jnp.int32, m2.shape, 1)
    m2 = m2 + (col == 1).astype(_f32)
    s2_ref[...] = m2[:, :32]
    d2_ref[...] = m2[:, 32:]


def _post_body(a0_ref, a1_ref, b2_ref, o_ref):
    num = a0_ref[:, 0:1] + a1_ref[:, 0:1]
    den = a0_ref[:, 1:2] + a1_ref[:, 1:2]
    o_ref[...] = num / (den + 1e-16) + b2_ref[...]


def _sc_edge_kernel(chunks, epw, s_cols, d_cols):
    lo = s_cols - d_cols

    mesh = plsc.VectorSubcoreMesh(core_axis_name="c", subcore_axis_name="s")

    @functools.partial(
        pl.kernel,
        out_type=jax.ShapeDtypeStruct((2 * NPAD, s_cols), _f32),
        mesh=mesh,
        compiler_params=pltpu.CompilerParams(use_tc_tiling_on_sc=False),
        scratch_types=[
            pltpu.VMEM((K,), jnp.int32),
            pltpu.VMEM((K,), jnp.int32),
            pltpu.VMEM((K, s_cols), _f32),
            pltpu.VMEM((K, d_cols), _f32),
            pltpu.VMEM((K, s_cols), _f32),
            pltpu.VMEM_SHARED((NPAD, s_cols), _f32),
            pltpu.SemaphoreType.DMA,
            pltpu.SemaphoreType.DMA,
        ],
    )
    def k(src_hbm, dst_hbm, s_hbm, d_hbm, z_hbm, out_hbm,
          sidx, didx, srow, drow, mrow, acc, sem1, sem2):
        cid = lax.axis_index("c")
        sid = lax.axis_index("s")
        wid = sid * 2 + cid

        pltpu.sync_copy(z_hbm.at[pl.ds(sid * ROWS_PER_TILE, ROWS_PER_TILE)],
                        acc.at[pl.ds(sid * ROWS_PER_TILE, ROWS_PER_TILE)])
        plsc.subcore_barrier()

        base = wid * epw

        def chunk_body(kk, _):
            off = base + kk * K
            pltpu.sync_copy(src_hbm.at[pl.ds(off, K)], sidx)
            pltpu.sync_copy(dst_hbm.at[pl.ds(off, K)], didx)
            cp1 = pltpu.async_copy(s_hbm.at[sidx], srow, sem1)
            cp2 = pltpu.async_copy(d_hbm.at[didx], drow, sem2)
            cp1.wait()
            cp2.wait()

            def edge_body(e, _):
                for j in range(d_cols // 16):
                    a = srow[e, pl.ds(lo + 16 * j, 16)] + drow[e, pl.ds(16 * j, 16)]
                    a = jnp.where(a > 0, a, 0.2 * a)
                    al = jnp.exp(a)
                    mrow[e, pl.ds(lo + 16 * j, 16)] = al
                    mrow[e, pl.ds(16 * j, 16)] = srow[e, pl.ds(16 * j, 16)] * al
                return 0

            lax.fori_loop(0, K, edge_body, 0)
            pltpu.sync_copy(mrow, acc.at[didx], add=True)
            return 0

        lax.fori_loop(0, chunks, chunk_body, 0)
        plsc.subcore_barrier()
        pltpu.sync_copy(
            acc.at[pl.ds(sid * ROWS_PER_TILE, ROWS_PER_TILE)],
            out_hbm.at[pl.ds(cid * NPAD + sid * ROWS_PER_TILE, ROWS_PER_TILE)])

    return k


def _sc2_edge_kernel(chunks, epw):
    mesh = plsc.VectorSubcoreMesh(core_axis_name="c", subcore_axis_name="s")

    @functools.partial(
        pl.kernel,
        out_type=jax.ShapeDtypeStruct((2 * NPAD, 16), _f32),
        mesh=mesh,
        compiler_params=pltpu.CompilerParams(use_tc_tiling_on_sc=False),
        scratch_types=[
            pltpu.VMEM((K,), jnp.int32),
            pltpu.VMEM((K,), jnp.int32),
            pltpu.VMEM((K, 32), _f32),
            pltpu.VMEM((K, 16), _f32),
            pltpu.VMEM((K, 16), _f32),
            pltpu.VMEM_SHARED((NPAD, 16), _f32),
            pltpu.SemaphoreType.DMA,
            pltpu.SemaphoreType.DMA,
        ],
    )
    def k(src_hbm, dst_hbm, s_hbm, d_hbm, z_hbm, out_hbm,
          sidx, didx, srow, drow, mrow, acc, sem1, sem2):
        cid = lax.axis_index("c")
        sid = lax.axis_index("s")
        wid = sid * 2 + cid

        pltpu.sync_copy(z_hbm.at[pl.ds(sid * ROWS_PER_TILE, ROWS_PER_TILE)],
                        acc.at[pl.ds(sid * ROWS_PER_TILE, ROWS_PER_TILE)])
        plsc.subcore_barrier()

        base = wid * epw

        def chunk_body(kk, _):
            off = base + kk * K
            pltpu.sync_copy(src_hbm.at[pl.ds(off, K)], sidx)
            pltpu.sync_copy(dst_hbm.at[pl.ds(off, K)], didx)
            cp1 = pltpu.async_copy(s_hbm.at[sidx], srow, sem1)
            cp2 = pltpu.async_copy(d_hbm.at[didx], drow, sem2)
            cp1.wait()
            cp2.wait()

            def edge_body(e, _):
                a = srow[e, pl.ds(16, 16)] + drow[e, pl.ds(0, 16)]
                a = jnp.where(a > 0, a, 0.2 * a)
                al = jnp.exp(a)
                mrow[e, pl.ds(0, 16)] = srow[e, pl.ds(0, 16)] * al
                return 0

            lax.fori_loop(0, K, edge_body, 0)
            pltpu.sync_copy(mrow, acc.at[didx], add=True)
            return 0

        lax.fori_loop(0, chunks, chunk_body, 0)
        plsc.subcore_barrier()
        pltpu.sync_copy(
            acc.at[pl.ds(sid * ROWS_PER_TILE, ROWS_PER_TILE)],
            out_hbm.at[pl.ds(cid * NPAD + sid * ROWS_PER_TILE, ROWS_PER_TILE)])

    return k


def kernel(x, edge_index, W1, att_src1, att_dst1, b1, W2, att_src2, att_dst2, b2):
    n_edges = edge_index.shape[1]
    etot, epw, chunks = _edge_layout(n_edges)

    loop = jnp.arange(N_NODES, dtype=jnp.int32)
    npad_e = etot - n_edges - N_NODES
    src = jnp.concatenate([edge_index[0], loop, jnp.zeros((npad_e,), jnp.int32)])
    dst = jnp.concatenate([edge_index[1], loop,
                           jnp.full((npad_e,), N_NODES, jnp.int32)])

    eye8 = jnp.eye(8, dtype=_f32)

    def dupmat(a):
        m = a[:, :, None, None] * eye8[:, None, :, None]
        return jnp.broadcast_to(m, (8, 8, 8, 8)).reshape(64, 64)

    Wall = jnp.concatenate(
        [W1, W1 @ dupmat(att_src1[0]), W1 @ dupmat(att_dst1[0])], axis=1)
    x_pad = jnp.concatenate([x, jnp.zeros((NPAD - N_NODES, D_IN), _f32)])

    BR = 1024
    grid = NPAD // BR
    S, D = pl.pallas_call(
        _pre_body,
        grid=(grid,),
        in_specs=[
            pl.BlockSpec((BR, 128), lambda i: (i, 0)),
            pl.BlockSpec((128, 192), lambda i: (0, 0)),
        ],
        out_specs=[
            pl.BlockSpec((BR, 128), lambda i: (i, 0)),
            pl.BlockSpec((BR, 64), lambda i: (i, 0)),
        ],
        out_shape=[
            jax.ShapeDtypeStruct((NPAD, 128), _f32),
            jax.ShapeDtypeStruct((NPAD, 64), _f32),
        ],
    )(x_pad, Wall)

    z1 = jnp.zeros((NPAD, 128), _f32)
    acc1 = _sc_edge_kernel(chunks, epw, 128, 64)(src, dst, S, D, z1)

    sc2 = att_src2[0, 0, 0]
    dc2 = att_dst2[0, 0, 0]
    W2e = jnp.concatenate(
        [W2, jnp.zeros((64, 15), _f32),
         jnp.tile(W2 * sc2, (1, 16)), jnp.tile(W2 * dc2, (1, 16))], axis=1)
    b1r = b1.reshape(1, 64)

    S2, D2 = pl.pallas_call(
        _mid_body,
        grid=(grid,),
        in_specs=[
            pl.BlockSpec((BR, 128), lambda i: (i, 0)),
            pl.BlockSpec((BR, 128), lambda i: (grid + i, 0)),
            pl.BlockSpec((1, 64), lambda i: (0, 0)),
            pl.BlockSpec((64, 48), lambda i: (0, 0)),
        ],
        out_specs=[
            pl.BlockSpec((BR, 32), lambda i: (i, 0)),
            pl.BlockSpec((BR, 16), lambda i: (i, 0)),
        ],
        out_shape=[
            jax.ShapeDtypeStruct((NPAD, 32), _f32),
            jax.ShapeDtypeStruct((NPAD, 16), _f32),
        ],
    )(acc1, acc1, b1r, W2e)

    z2 = jnp.zeros((NPAD, 16), _f32)
    acc2 = _sc2_edge_kernel(chunks, epw)(src, dst, S2, D2, z2)

    b2r = jnp.broadcast_to(b2.reshape(1, 1), (1, 16))
    out = pl.pallas_call(
        _post_body,
        grid=(grid,),
        in_specs=[
            pl.BlockSpec((BR, 16), lambda i: (i, 0)),
            pl.BlockSpec((BR, 16), lambda i: (grid + i, 0)),
            pl.BlockSpec((1, 16), lambda i: (0, 0)),
        ],
        out_specs=pl.BlockSpec((BR, 16), lambda i: (i, 0)),
        out_shape=jax.ShapeDtypeStruct((NPAD, 16), _f32),
    )(acc2, acc2, b2r)

    return out[:N_NODES, :1]
